# baseline (speedup 1.0000x reference)
import functools

import jax
import jax.numpy as jnp
from jax import lax
from jax.experimental import pallas as pl
from jax.experimental.pallas import tpu as pltpu
from jax.experimental.pallas import tpu_sc as plsc

B = 16
T = 2048
D = 64
NH = 8
NB = 32
TOTAL = NH * T
NEG = -50000.0
NW = 32
ROWS_PER_W = (B * NH) // NW
SLOTS_PER_W = (B * TOTAL) // NW


def _hash_pos_body(qk_ref, rot_ref, plocal_ref, p_ref, pg_ref):
    b = pl.program_id(0)
    h = pl.program_id(1)
    qkb = qk_ref[0]
    r = rot_ref[0]
    rot = lax.dot_general(qkb, r, (((1,), (0,)), ((), ())),
                          preferred_element_type=jnp.float32)
    x = jnp.concatenate([rot, -rot], axis=1)
    m = jnp.max(x, axis=1, keepdims=True)
    iota = lax.broadcasted_iota(jnp.int32, (T, NB), 1)
    bi = jnp.min(jnp.where(x == m, iota, NB), axis=1, keepdims=True)
    oh = (iota == bi).astype(jnp.float32)
    tri = (lax.broadcasted_iota(jnp.int32, (128, 128), 0)
           >= lax.broadcasted_iota(jnp.int32, (128, 128), 1)).astype(jnp.float32)
    run = jnp.zeros((1, NB), jnp.float32)
    ranks = []
    for j in range(T // 128):
        blk = oh[j * 128:(j + 1) * 128, :]
        cg = lax.dot_general(tri, blk, (((1,), (0,)), ((), ())),
                             preferred_element_type=jnp.float32) + run
        ranks.append(jnp.sum(cg * blk, axis=1, keepdims=True) - 1.0)
        run = run + jnp.sum(blk, axis=0, keepdims=True)
    rank = jnp.concatenate(ranks, axis=0)
    startsel = jnp.sum(jnp.where(iota < bi, run, 0.0), axis=1, keepdims=True)
    plocal = (startsel + rank).astype(jnp.int32)
    plocal_ref[0] = plocal
    p_ref[0] = plocal + h * T
    pg_ref[0] = plocal + h * T + b * TOTAL


def _hash_pos(qk2, rot):
    return pl.pallas_call(
        _hash_pos_body,
        grid=(B, NH),
        in_specs=[
            pl.BlockSpec((1, T, D), lambda b, h: (b, 0, 0)),
            pl.BlockSpec((1, D, NB // 2), lambda b, h: (h, 0, 0)),
        ],
        out_specs=[pl.BlockSpec((1, T, 1), lambda b, h: (b * NH + h, 0, 0))] * 3,
        out_shape=[jax.ShapeDtypeStruct((B * NH, T, 1), jnp.int32)] * 3,
    )(qk2, rot)


@functools.lru_cache(maxsize=None)
def _mesh():
    return plsc.VectorSubcoreMesh(core_axis_name="c", subcore_axis_name="s")


@functools.lru_cache(maxsize=None)
def _sc_scatter_st():
    return pl.kernel(
        _sc_scatter_st_body,
        compiler_params=pltpu.CompilerParams(needs_layout_passes=False, use_tc_tiling_on_sc=False),
        out_type=[jax.ShapeDtypeStruct((B * NH, T), jnp.int32),
                  jax.ShapeDtypeStruct((B * NH, T), jnp.int32)],
        mesh=_mesh(),
        scratch_types=[pltpu.VMEM((T,), jnp.int32),
                       pltpu.VMEM((T,), jnp.int32),
                       pltpu.VMEM((T,), jnp.int32)],
    )


def _sc_scatter_st_body(pl_hbm, st_hbm, gst_hbm, pv, stv, gstv):
    wid = lax.axis_index("s") * 2 + lax.axis_index("c")

    def row_body(rr, carry):
        r = wid * ROWS_PER_W + rr
        b = r // NH
        pltpu.sync_copy(pl_hbm.at[r], pv)

        def inner(j, carry2):
            idx = pv[pl.ds(j * 16, 16)]
            t = j * 16 + lax.iota(jnp.int32, 16)
            plsc.store_scatter(stv, [idx], t)
            plsc.store_scatter(gstv, [idx], t + b * T)
            return carry2

        lax.fori_loop(0, T // 16, inner, 0)
        pltpu.sync_copy(stv, st_hbm.at[r])
        pltpu.sync_copy(gstv, gst_hbm.at[r])
        return carry

    lax.fori_loop(0, ROWS_PER_W, row_body, 0)


@functools.lru_cache(maxsize=None)
def _sc_gather_qkv():
    return pl.kernel(
        _sc_gather_qkv_body,
        compiler_params=pltpu.CompilerParams(needs_layout_passes=False, use_tc_tiling_on_sc=False),
        out_type=[jax.ShapeDtypeStruct((B * TOTAL, D), jnp.float32),
                  jax.ShapeDtypeStruct((B * TOTAL, D), jnp.float32)],
        mesh=_mesh(),
        scratch_types=[pltpu.VMEM((SLOTS_PER_W // 128, 128), jnp.int32),
                       pltpu.VMEM((1024, D), jnp.float32),
                       pltpu.SemaphoreType.DMA],
    )


def _sc_gather_qkv_body(qk_hbm, v_hbm, gst_hbm, sqk_hbm, sv_hbm, idx2, rows, sem):
    wid = lax.axis_index("s") * 2 + lax.axis_index("c")
    pltpu.sync_copy(gst_hbm.at[wid], idx2)
    base = wid * SLOTS_PER_W

    def chunk(cc, carry):
        s0 = base + cc * 1024
        hs = [pltpu.async_copy(qk_hbm.at[idx2.at[cc * 8 + kk]],
                               rows.at[pl.ds(kk * 128, 128)], sem)
              for kk in range(8)]
        for hh in hs:
            hh.wait()
        pltpu.sync_copy(rows, sqk_hbm.at[pl.ds(s0, 1024)])
        hs = [pltpu.async_copy(v_hbm.at[idx2.at[cc * 8 + kk]],
                               rows.at[pl.ds(kk * 128, 128)], sem)
              for kk in range(8)]
        for hh in hs:
            hh.wait()
        pltpu.sync_copy(rows, sv_hbm.at[pl.ds(s0, 1024)])
        return carry

    lax.fori_loop(0, SLOTS_PER_W // 1024, chunk, 0)


def _att_body(q_ref, kh_ref, v_ref, vh_ref, tr_ref, th_ref, tc_ref,
              so_ref, sl_ref, nk_s, vx_s, mb_s):
    qall = q_ref[0]
    norm = jnp.sqrt(jnp.sum(qall * qall, axis=1, keepdims=True))
    nk_s[64:, :] = qall / jnp.maximum(norm, 1e-12)
    kh = kh_ref[0]
    nhh = jnp.sqrt(jnp.sum(kh * kh, axis=1, keepdims=True))
    nk_s[0:64, :] = kh / jnp.maximum(nhh, 1e-12)
    vx_s[64:, :] = v_ref[0]
    vx_s[0:64, :] = vh_ref[0]
    trow = tr_ref[...].reshape(32, 64)
    tprev = jnp.concatenate([th_ref[...].reshape(1, 64), trow[:31, :]], axis=0)
    tcol = tc_ref[0]
    tkb = jnp.concatenate([tprev, trow], axis=1)
    tkbig = jnp.broadcast_to(tkb[:, None, :], (32, 64, 128)).reshape(T, 128)
    mb_s[...] = (tcol == tkbig).astype(jnp.float32)

    def body(c, carry):
        q = q_ref[0, pl.ds(c * 64, 64), :]
        kb = nk_s[pl.ds(c * 64, 128), :]
        d = lax.dot_general(q, kb, (((1,), (1,)), ((), ())),
                            preferred_element_type=jnp.float32)
        d = jnp.where(mb_s[pl.ds(c * 64, 64), :] != 0, NEG, d)
        m = jnp.max(d, axis=1, keepdims=True)
        e = jnp.exp(d - m)
        s = jnp.sum(e, axis=1, keepdims=True)
        vb = vx_s[pl.ds(c * 64, 128), :]
        o = lax.dot_general(e, vb, (((1,), (0,)), ((), ())),
                            preferred_element_type=jnp.float32) / s
        so_ref[0, pl.ds(c * 64, 64), :] = o
        sl_ref[0, pl.ds(c * 64, 64), :] = m + jnp.log(s)
        return carry

    lax.fori_loop(0, 32, body, 0)


def _attention(sqk, sv, trow, tcol):
    halo = lambda b, i: (b, (32 * i + 255) % 256, 0)
    cur = lambda b, i: (b, i, 0)
    halo4 = lambda b, i: (b, (32 * i + 255) % 256, 0, 0)
    cur4 = lambda b, i: (b, i, 0, 0)
    return pl.pallas_call(
        _att_body,
        grid=(B, TOTAL // 2048),
        in_specs=[
            pl.BlockSpec((1, 2048, D), cur),
            pl.BlockSpec((1, 64, D), halo),
            pl.BlockSpec((1, 2048, D), cur),
            pl.BlockSpec((1, 64, D), halo),
            pl.BlockSpec((1, 32, 1, 64), cur4),
            pl.BlockSpec((1, 1, 1, 64), halo4),
            pl.BlockSpec((1, 2048, 1), cur),
        ],
        out_specs=[pl.BlockSpec((1, 2048, D), cur),
                   pl.BlockSpec((1, 2048, 1), cur)],
        out_shape=[jax.ShapeDtypeStruct((B, TOTAL, D), jnp.float32),
                   jax.ShapeDtypeStruct((B, TOTAL, 1), jnp.float32)],
        scratch_shapes=[pltpu.VMEM((2112, D), jnp.float32),
                        pltpu.VMEM((2112, D), jnp.float32),
                        pltpu.VMEM((T, 128), jnp.float32)],
    )(sqk, sqk, sv, sv, trow, trow, tcol)


@functools.lru_cache(maxsize=None)
def _sc_unsort():
    return pl.kernel(
        _sc_unsort_body,
        compiler_params=pltpu.CompilerParams(needs_layout_passes=False, use_tc_tiling_on_sc=False),
        out_type=[jax.ShapeDtypeStruct((B * TOTAL, D), jnp.float32),
                  jax.ShapeDtypeStruct((B * TOTAL,), jnp.float32)],
        mesh=_mesh(),
        scratch_types=[pltpu.VMEM((SLOTS_PER_W // 128, 128), jnp.int32),
                       pltpu.VMEM((SLOTS_PER_W,), jnp.int32),
                       pltpu.VMEM((TOTAL,), jnp.float32),
                       pltpu.VMEM((1024, D), jnp.float32),
                       pltpu.VMEM((SLOTS_PER_W,), jnp.float32),
                       pltpu.SemaphoreType.DMA],
    )


def _sc_unsort_body(so_hbm, slog_hbm, pg_hbm, p_hbm, of_hbm, lf_hbm,
                    idx2, pv, slogv, rows, lout, sem):
    wid = lax.axis_index("s") * 2 + lax.axis_index("c")
    b = wid // 2
    base = wid * SLOTS_PER_W
    pltpu.sync_copy(pg_hbm.at[wid], idx2)
    pltpu.sync_copy(p_hbm.at[pl.ds(base, SLOTS_PER_W)], pv)
    pltpu.sync_copy(slog_hbm.at[b], slogv)

    def chunk(cc, carry):
        s0 = base + cc * 1024
        hs = [pltpu.async_copy(so_hbm.at[idx2.at[cc * 8 + kk]],
                               rows.at[pl.ds(kk * 128, 128)], sem)
              for kk in range(8)]
        for hh in hs:
            hh.wait()
        pltpu.sync_copy(rows, of_hbm.at[pl.ds(s0, 1024)])
        return carry

    lax.fori_loop(0, SLOTS_PER_W // 1024, chunk, 0)

    def lchunk(j, carry):
        idx16 = pv[pl.ds(j * 16, 16)]
        lout[pl.ds(j * 16, 16)] = plsc.load_gather(slogv, [idx16])
        return carry

    lax.fori_loop(0, SLOTS_PER_W // 16, lchunk, 0)
    pltpu.sync_copy(lout, lf_hbm.at[pl.ds(base, SLOTS_PER_W)])


def _combine_body(o_ref, l_ref, out_ref):
    ls = [l_ref[0, hh] for hh in range(NH)]
    m = ls[0]
    for hh in range(1, NH):
        m = jnp.maximum(m, ls[hh])
    es = [jnp.exp(lh - m) for lh in ls]
    s = es[0]
    for hh in range(1, NH):
        s = s + es[hh]
    acc = o_ref[0, 0] * es[0]
    for hh in range(1, NH):
        acc = acc + o_ref[0, hh] * es[hh]
    out_ref[0] = acc / s


def _combine(of, lf):
    return pl.pallas_call(
        _combine_body,
        grid=(B,),
        in_specs=[pl.BlockSpec((1, NH, T, D), lambda b: (b, 0, 0, 0)),
                  pl.BlockSpec((1, NH, T, 1), lambda b: (b, 0, 0, 0))],
        out_specs=pl.BlockSpec((1, T, D), lambda b: (b, 0, 0)),
        out_shape=jax.ShapeDtypeStruct((B, T, D), jnp.float32),
    )(of, lf)


def kernel(qk, k, v):
    del k
    qk2 = jnp.transpose(qk, (0, 2, 1, 3)).reshape(B, T, D)
    v2 = jnp.transpose(v, (0, 2, 1, 3)).reshape(B, T, D)
    rot = jax.random.normal(jax.random.key(42), (1, D, NH, NB // 2),
                            jnp.float32)[0].transpose(1, 0, 2)
    plocal3, p3, pg3 = _hash_pos(qk2, rot)
    st2, gst2 = _sc_scatter_st()(plocal3.reshape(B * NH, T))
    sqk, sv = _sc_gather_qkv()(qk2.reshape(B * T, D), v2.reshape(B * T, D),
                               gst2.reshape(NW, SLOTS_PER_W // 128, 128))
    stf = st2.reshape(B, TOTAL).astype(jnp.float32)
    so, slog = _attention(sqk.reshape(B, TOTAL, D), sv.reshape(B, TOTAL, D),
                          stf.reshape(B, TOTAL // 64, 1, 64),
                          stf.reshape(B, TOTAL, 1))
    of, lf = _sc_unsort()(so.reshape(B * TOTAL, D), slog.reshape(B, TOTAL),
                          pg3.reshape(NW, SLOTS_PER_W // 128, 128),
                          p3.reshape(B * TOTAL))
    return _combine(of.reshape(B, NH, T, D), lf.reshape(B, NH, T, 1))


# speedup vs baseline: 1.1351x; 1.1351x over previous
import functools

import jax
import jax.numpy as jnp
from jax import lax
from jax.experimental import pallas as pl
from jax.experimental.pallas import tpu as pltpu
from jax.experimental.pallas import tpu_sc as plsc

B = 16
T = 2048
D = 64
NH = 8
NB = 32
TOTAL = NH * T
NEG = -50000.0
NW = 32
ROWS_PER_W = (B * NH) // NW
SLOTS_PER_W = (B * TOTAL) // NW


def _hash_pos_body(qk_ref, rot_ref, plocal_ref, p_ref, pg_ref):
    b = pl.program_id(0)
    h = pl.program_id(1)
    qkb = qk_ref[0]
    r = rot_ref[0]
    rot = lax.dot_general(qkb, r, (((1,), (0,)), ((), ())),
                          preferred_element_type=jnp.float32)
    x = jnp.concatenate([rot, -rot], axis=1)
    m = jnp.max(x, axis=1, keepdims=True)
    iota = lax.broadcasted_iota(jnp.int32, (T, NB), 1)
    bi = jnp.min(jnp.where(x == m, iota, NB), axis=1, keepdims=True)
    oh = (iota == bi).astype(jnp.float32)
    tri = (lax.broadcasted_iota(jnp.int32, (128, 128), 0)
           >= lax.broadcasted_iota(jnp.int32, (128, 128), 1)).astype(jnp.float32)
    run = jnp.zeros((1, NB), jnp.float32)
    ranks = []
    for j in range(T // 128):
        blk = oh[j * 128:(j + 1) * 128, :]
        cg = lax.dot_general(tri, blk, (((1,), (0,)), ((), ())),
                             preferred_element_type=jnp.float32) + run
        ranks.append(jnp.sum(cg * blk, axis=1, keepdims=True) - 1.0)
        run = run + jnp.sum(blk, axis=0, keepdims=True)
    rank = jnp.concatenate(ranks, axis=0)
    startsel = jnp.sum(jnp.where(iota < bi, run, 0.0), axis=1, keepdims=True)
    plocal = (startsel + rank).astype(jnp.int32)
    plocal_ref[0] = plocal
    p_ref[0] = plocal + h * T
    pg_ref[0] = plocal + h * T + b * TOTAL


def _hash_pos(qk2, rot):
    return pl.pallas_call(
        _hash_pos_body,
        grid=(B, NH),
        in_specs=[
            pl.BlockSpec((1, T, D), lambda b, h: (b, 0, 0)),
            pl.BlockSpec((1, D, NB // 2), lambda b, h: (h, 0, 0)),
        ],
        out_specs=[pl.BlockSpec((1, T, 1), lambda b, h: (b * NH + h, 0, 0))] * 3,
        out_shape=[jax.ShapeDtypeStruct((B * NH, T, 1), jnp.int32)] * 3,
    )(qk2, rot)


@functools.lru_cache(maxsize=None)
def _mesh():
    return plsc.VectorSubcoreMesh(core_axis_name="c", subcore_axis_name="s")


@functools.lru_cache(maxsize=None)
def _sc_scatter_st():
    return pl.kernel(
        _sc_scatter_st_body,
        compiler_params=pltpu.CompilerParams(needs_layout_passes=False, use_tc_tiling_on_sc=False),
        out_type=[jax.ShapeDtypeStruct((B * NH, T), jnp.int32),
                  jax.ShapeDtypeStruct((B * NH, T), jnp.int32)],
        mesh=_mesh(),
        scratch_types=[pltpu.VMEM((T,), jnp.int32),
                       pltpu.VMEM((T,), jnp.int32),
                       pltpu.VMEM((T,), jnp.int32)],
    )


def _sc_scatter_st_body(pl_hbm, st_hbm, gst_hbm, pv, stv, gstv):
    wid = lax.axis_index("s") * 2 + lax.axis_index("c")

    def row_body(rr, carry):
        r = wid * ROWS_PER_W + rr
        b = r // NH
        pltpu.sync_copy(pl_hbm.at[r], pv)

        def inner(j, carry2):
            idx = pv[pl.ds(j * 16, 16)]
            t = j * 16 + lax.iota(jnp.int32, 16)
            plsc.store_scatter(stv, [idx], t)
            plsc.store_scatter(gstv, [idx], t + b * T)
            return carry2

        lax.fori_loop(0, T // 16, inner, 0)
        pltpu.sync_copy(stv, st_hbm.at[r])
        pltpu.sync_copy(gstv, gst_hbm.at[r])
        return carry

    lax.fori_loop(0, ROWS_PER_W, row_body, 0)


@functools.lru_cache(maxsize=None)
def _sc_gather_qkv():
    return pl.kernel(
        _sc_gather_qkv_body,
        compiler_params=pltpu.CompilerParams(needs_layout_passes=False, use_tc_tiling_on_sc=False),
        out_type=[jax.ShapeDtypeStruct((B * TOTAL, D), jnp.float32),
                  jax.ShapeDtypeStruct((B * TOTAL, D), jnp.float32)],
        mesh=_mesh(),
        scratch_types=[pltpu.VMEM((SLOTS_PER_W // 128, 128), jnp.int32),
                       pltpu.VMEM((1024, D), jnp.float32),
                       pltpu.SemaphoreType.DMA],
    )


def _sc_gather_qkv_body(qk_hbm, v_hbm, gst_hbm, sqk_hbm, sv_hbm, idx2, rows, sem):
    wid = lax.axis_index("s") * 2 + lax.axis_index("c")
    pltpu.sync_copy(gst_hbm.at[wid], idx2)
    base = wid * SLOTS_PER_W

    def chunk(cc, carry):
        s0 = base + cc * 1024
        hs = [pltpu.async_copy(qk_hbm.at[idx2.at[cc * 8 + kk]],
                               rows.at[pl.ds(kk * 128, 128)], sem)
              for kk in range(8)]
        for hh in hs:
            hh.wait()
        pltpu.sync_copy(rows, sqk_hbm.at[pl.ds(s0, 1024)])
        hs = [pltpu.async_copy(v_hbm.at[idx2.at[cc * 8 + kk]],
                               rows.at[pl.ds(kk * 128, 128)], sem)
              for kk in range(8)]
        for hh in hs:
            hh.wait()
        pltpu.sync_copy(rows, sv_hbm.at[pl.ds(s0, 1024)])
        return carry

    lax.fori_loop(0, SLOTS_PER_W // 1024, chunk, 0)


def _att_body(q_ref, kh_ref, v_ref, vh_ref, tr_ref, th_ref, tc_ref,
              so_ref, sl_ref, nk_s, vx_s, mb_s):
    qall = q_ref[0]
    norm = jnp.sqrt(jnp.sum(qall * qall, axis=1, keepdims=True))
    nk_s[64:, :] = qall / jnp.maximum(norm, 1e-12)
    kh = kh_ref[0]
    nhh = jnp.sqrt(jnp.sum(kh * kh, axis=1, keepdims=True))
    nk_s[0:64, :] = kh / jnp.maximum(nhh, 1e-12)
    vx_s[64:, :] = v_ref[0]
    vx_s[0:64, :] = vh_ref[0]
    trow = tr_ref[...].reshape(32, 64)
    tprev = jnp.concatenate([th_ref[...].reshape(1, 64), trow[:31, :]], axis=0)
    tcol = tc_ref[0]
    tkb = jnp.concatenate([tprev, trow], axis=1)
    tkbig = jnp.broadcast_to(tkb[:, None, :], (32, 64, 128)).reshape(T, 128)
    mb_s[...] = (tcol == tkbig).astype(jnp.float32)

    def body(c, carry):
        q = q_ref[0, pl.ds(c * 64, 64), :]
        kb = nk_s[pl.ds(c * 64, 128), :]
        d = lax.dot_general(q, kb, (((1,), (1,)), ((), ())),
                            preferred_element_type=jnp.float32)
        d = jnp.where(mb_s[pl.ds(c * 64, 64), :] != 0, NEG, d)
        m = jnp.max(d, axis=1, keepdims=True)
        e = jnp.exp(d - m)
        s = jnp.sum(e, axis=1, keepdims=True)
        vb = vx_s[pl.ds(c * 64, 128), :]
        o = lax.dot_general(e, vb, (((1,), (0,)), ((), ())),
                            preferred_element_type=jnp.float32) / s
        so_ref[0, pl.ds(c * 64, 64), :] = o
        sl_ref[0, pl.ds(c * 64, 64), :] = m + jnp.log(s)
        return carry

    lax.fori_loop(0, 32, body, 0, unroll=4)


def _attention(sqk, sv, trow, tcol):
    halo = lambda b, i: (b, (32 * i + 255) % 256, 0)
    cur = lambda b, i: (b, i, 0)
    halo4 = lambda b, i: (b, (32 * i + 255) % 256, 0, 0)
    cur4 = lambda b, i: (b, i, 0, 0)
    return pl.pallas_call(
        _att_body,
        grid=(B, TOTAL // 2048),
        in_specs=[
            pl.BlockSpec((1, 2048, D), cur),
            pl.BlockSpec((1, 64, D), halo),
            pl.BlockSpec((1, 2048, D), cur),
            pl.BlockSpec((1, 64, D), halo),
            pl.BlockSpec((1, 32, 1, 64), cur4),
            pl.BlockSpec((1, 1, 1, 64), halo4),
            pl.BlockSpec((1, 2048, 1), cur),
        ],
        out_specs=[pl.BlockSpec((1, 2048, D), cur),
                   pl.BlockSpec((1, 2048, 1), cur)],
        out_shape=[jax.ShapeDtypeStruct((B, TOTAL, D), jnp.float32),
                   jax.ShapeDtypeStruct((B, TOTAL, 1), jnp.float32)],
        scratch_shapes=[pltpu.VMEM((2112, D), jnp.float32),
                        pltpu.VMEM((2112, D), jnp.float32),
                        pltpu.VMEM((T, 128), jnp.float32)],
    )(sqk, sqk, sv, sv, trow, trow, tcol)


@functools.lru_cache(maxsize=None)
def _sc_unsort():
    return pl.kernel(
        _sc_unsort_body,
        compiler_params=pltpu.CompilerParams(needs_layout_passes=False, use_tc_tiling_on_sc=False),
        out_type=[jax.ShapeDtypeStruct((B * TOTAL, D), jnp.float32),
                  jax.ShapeDtypeStruct((B * TOTAL,), jnp.float32)],
        mesh=_mesh(),
        scratch_types=[pltpu.VMEM((SLOTS_PER_W // 128, 128), jnp.int32),
                       pltpu.VMEM((SLOTS_PER_W,), jnp.int32),
                       pltpu.VMEM((TOTAL,), jnp.float32),
                       pltpu.VMEM((1024, D), jnp.float32),
                       pltpu.VMEM((SLOTS_PER_W,), jnp.float32),
                       pltpu.SemaphoreType.DMA],
    )


def _sc_unsort_body(so_hbm, slog_hbm, pg_hbm, p_hbm, of_hbm, lf_hbm,
                    idx2, pv, slogv, rows, lout, sem):
    wid = lax.axis_index("s") * 2 + lax.axis_index("c")
    b = wid // 2
    base = wid * SLOTS_PER_W
    pltpu.sync_copy(pg_hbm.at[wid], idx2)
    pltpu.sync_copy(p_hbm.at[pl.ds(base, SLOTS_PER_W)], pv)
    pltpu.sync_copy(slog_hbm.at[b], slogv)

    def chunk(cc, carry):
        s0 = base + cc * 1024
        hs = [pltpu.async_copy(so_hbm.at[idx2.at[cc * 8 + kk]],
                               rows.at[pl.ds(kk * 128, 128)], sem)
              for kk in range(8)]
        for hh in hs:
            hh.wait()
        pltpu.sync_copy(rows, of_hbm.at[pl.ds(s0, 1024)])
        return carry

    lax.fori_loop(0, SLOTS_PER_W // 1024, chunk, 0)

    def lchunk(j, carry):
        idx16 = pv[pl.ds(j * 16, 16)]
        lout[pl.ds(j * 16, 16)] = plsc.load_gather(slogv, [idx16])
        return carry

    lax.fori_loop(0, SLOTS_PER_W // 16, lchunk, 0)
    pltpu.sync_copy(lout, lf_hbm.at[pl.ds(base, SLOTS_PER_W)])


def _combine_body(o_ref, l_ref, out_ref):
    ls = [l_ref[0, hh] for hh in range(NH)]
    m = ls[0]
    for hh in range(1, NH):
        m = jnp.maximum(m, ls[hh])
    es = [jnp.exp(lh - m) for lh in ls]
    s = es[0]
    for hh in range(1, NH):
        s = s + es[hh]
    acc = o_ref[0, 0] * es[0]
    for hh in range(1, NH):
        acc = acc + o_ref[0, hh] * es[hh]
    out_ref[0] = acc / s


def _combine(of, lf):
    return pl.pallas_call(
        _combine_body,
        grid=(B,),
        in_specs=[pl.BlockSpec((1, NH, T, D), lambda b: (b, 0, 0, 0)),
                  pl.BlockSpec((1, NH, T, 1), lambda b: (b, 0, 0, 0))],
        out_specs=pl.BlockSpec((1, T, D), lambda b: (b, 0, 0)),
        out_shape=jax.ShapeDtypeStruct((B, T, D), jnp.float32),
    )(of, lf)


def kernel(qk, k, v):
    del k
    qk2 = jnp.transpose(qk, (0, 2, 1, 3)).reshape(B, T, D)
    v2 = jnp.transpose(v, (0, 2, 1, 3)).reshape(B, T, D)
    rot = jax.random.normal(jax.random.key(42), (1, D, NH, NB // 2),
                            jnp.float32)[0].transpose(1, 0, 2)
    plocal3, p3, pg3 = _hash_pos(qk2, rot)
    st2, gst2 = _sc_scatter_st()(plocal3.reshape(B * NH, T))
    sqk, sv = _sc_gather_qkv()(qk2.reshape(B * T, D), v2.reshape(B * T, D),
                               gst2.reshape(NW, SLOTS_PER_W // 128, 128))
    stf = st2.reshape(B, TOTAL).astype(jnp.float32)
    so, slog = _attention(sqk.reshape(B, TOTAL, D), sv.reshape(B, TOTAL, D),
                          stf.reshape(B, TOTAL // 64, 1, 64),
                          stf.reshape(B, TOTAL, 1))
    of, lf = _sc_unsort()(so.reshape(B * TOTAL, D), slog.reshape(B, TOTAL),
                          pg3.reshape(NW, SLOTS_PER_W // 128, 128),
                          p3.reshape(B * TOTAL))
    return _combine(of.reshape(B, NH, T, D), lf.reshape(B, NH, T, 1))


# speedup vs baseline: 1.1742x; 1.0344x over previous
import functools

import jax
import jax.numpy as jnp
from jax import lax
from jax.experimental import pallas as pl
from jax.experimental.pallas import tpu as pltpu
from jax.experimental.pallas import tpu_sc as plsc

B = 16
T = 2048
D = 64
NH = 8
NB = 32
TOTAL = NH * T
NEG = -50000.0
NW = 32
ROWS_PER_W = (B * NH) // NW
SLOTS_PER_W = (B * TOTAL) // NW


def _hash_pos_body(qk_ref, rot_ref, plocal_ref, p_ref, pg_ref):
    b = pl.program_id(0)
    h = pl.program_id(1)
    qkb = qk_ref[0]
    r = rot_ref[0]
    rot = lax.dot_general(qkb, r, (((1,), (0,)), ((), ())),
                          preferred_element_type=jnp.float32)
    x = jnp.concatenate([rot, -rot], axis=1)
    m = jnp.max(x, axis=1, keepdims=True)
    iota = lax.broadcasted_iota(jnp.int32, (T, NB), 1)
    bi = jnp.min(jnp.where(x == m, iota, NB), axis=1, keepdims=True)
    oh = (iota == bi).astype(jnp.float32)
    tri = (lax.broadcasted_iota(jnp.int32, (128, 128), 0)
           >= lax.broadcasted_iota(jnp.int32, (128, 128), 1)).astype(jnp.float32)
    run = jnp.zeros((1, NB), jnp.float32)
    ranks = []
    for j in range(T // 128):
        blk = oh[j * 128:(j + 1) * 128, :]
        cg = lax.dot_general(tri, blk, (((1,), (0,)), ((), ())),
                             preferred_element_type=jnp.float32) + run
        ranks.append(jnp.sum(cg * blk, axis=1, keepdims=True) - 1.0)
        run = run + jnp.sum(blk, axis=0, keepdims=True)
    rank = jnp.concatenate(ranks, axis=0)
    startsel = jnp.sum(jnp.where(iota < bi, run, 0.0), axis=1, keepdims=True)
    plocal = (startsel + rank).astype(jnp.int32)
    plocal_ref[0] = plocal
    p_ref[0] = plocal + h * T
    pg_ref[0] = plocal + h * T + b * TOTAL


def _hash_pos(qk2, rot):
    return pl.pallas_call(
        _hash_pos_body,
        grid=(B, NH),
        in_specs=[
            pl.BlockSpec((1, T, D), lambda b, h: (b, 0, 0)),
            pl.BlockSpec((1, D, NB // 2), lambda b, h: (h, 0, 0)),
        ],
        out_specs=[pl.BlockSpec((1, T, 1), lambda b, h: (b * NH + h, 0, 0))] * 3,
        out_shape=[jax.ShapeDtypeStruct((B * NH, T, 1), jnp.int32)] * 3,
    )(qk2, rot)


@functools.lru_cache(maxsize=None)
def _mesh():
    return plsc.VectorSubcoreMesh(core_axis_name="c", subcore_axis_name="s")


@functools.lru_cache(maxsize=None)
def _sc_scatter_st():
    return pl.kernel(
        _sc_scatter_st_body,
        compiler_params=pltpu.CompilerParams(needs_layout_passes=False, use_tc_tiling_on_sc=False),
        out_type=[jax.ShapeDtypeStruct((B * NH, T), jnp.int32),
                  jax.ShapeDtypeStruct((B * NH, T), jnp.int32)],
        mesh=_mesh(),
        scratch_types=[pltpu.VMEM((T,), jnp.int32),
                       pltpu.VMEM((T,), jnp.int32),
                       pltpu.VMEM((T,), jnp.int32)],
    )


def _sc_scatter_st_body(pl_hbm, st_hbm, gst_hbm, pv, stv, gstv):
    wid = lax.axis_index("s") * 2 + lax.axis_index("c")

    def row_body(rr, carry):
        r = wid * ROWS_PER_W + rr
        b = r // NH
        pltpu.sync_copy(pl_hbm.at[r], pv)

        def inner(j, carry2):
            idx = pv[pl.ds(j * 16, 16)]
            t = j * 16 + lax.iota(jnp.int32, 16)
            plsc.store_scatter(stv, [idx], t)
            plsc.store_scatter(gstv, [idx], t + b * T)
            return carry2

        lax.fori_loop(0, T // 16, inner, 0)
        pltpu.sync_copy(stv, st_hbm.at[r])
        pltpu.sync_copy(gstv, gst_hbm.at[r])
        return carry

    lax.fori_loop(0, ROWS_PER_W, row_body, 0)


@functools.lru_cache(maxsize=None)
def _sc_gather_qkv():
    return pl.kernel(
        _sc_gather_qkv_body,
        compiler_params=pltpu.CompilerParams(needs_layout_passes=False, use_tc_tiling_on_sc=False),
        out_type=[jax.ShapeDtypeStruct((B * TOTAL, D), jnp.float32),
                  jax.ShapeDtypeStruct((B * TOTAL, D), jnp.float32)],
        mesh=_mesh(),
        scratch_types=[pltpu.VMEM((SLOTS_PER_W // 128, 128), jnp.int32),
                       pltpu.VMEM((1024, D), jnp.float32),
                       pltpu.SemaphoreType.DMA],
    )


def _sc_gather_qkv_body(qk_hbm, v_hbm, gst_hbm, sqk_hbm, sv_hbm, idx2, rows, sem):
    wid = lax.axis_index("s") * 2 + lax.axis_index("c")
    pltpu.sync_copy(gst_hbm.at[wid], idx2)
    base = wid * SLOTS_PER_W

    def chunk(cc, carry):
        s0 = base + cc * 1024
        hs = [pltpu.async_copy(qk_hbm.at[idx2.at[cc * 8 + kk]],
                               rows.at[pl.ds(kk * 128, 128)], sem)
              for kk in range(8)]
        for hh in hs:
            hh.wait()
        pltpu.sync_copy(rows, sqk_hbm.at[pl.ds(s0, 1024)])
        hs = [pltpu.async_copy(v_hbm.at[idx2.at[cc * 8 + kk]],
                               rows.at[pl.ds(kk * 128, 128)], sem)
              for kk in range(8)]
        for hh in hs:
            hh.wait()
        pltpu.sync_copy(rows, sv_hbm.at[pl.ds(s0, 1024)])
        return carry

    lax.fori_loop(0, SLOTS_PER_W // 1024, chunk, 0)


def _att_body(q_ref, kh_ref, v_ref, vh_ref, tr_ref, th_ref, tc_ref,
              so_ref, sl_ref, nk_s, vx_s, mb_s):
    qall = q_ref[0]
    norm = jnp.sqrt(jnp.sum(qall * qall, axis=1, keepdims=True))
    nk_s[64:, :] = qall / jnp.maximum(norm, 1e-12)
    kh = kh_ref[0]
    nhh = jnp.sqrt(jnp.sum(kh * kh, axis=1, keepdims=True))
    nk_s[0:64, :] = kh / jnp.maximum(nhh, 1e-12)
    vx_s[64:, :] = v_ref[0]
    vx_s[0:64, :] = vh_ref[0]
    trow = tr_ref[...].reshape(32, 64)
    tprev = jnp.concatenate([th_ref[...].reshape(1, 64), trow[:31, :]], axis=0)
    tcol = tc_ref[0]
    tkb = jnp.concatenate([tprev, trow], axis=1)
    tkbig = jnp.broadcast_to(tkb[:, None, :], (32, 64, 128)).reshape(T, 128)
    mb_s[...] = (tcol == tkbig).astype(jnp.float32)

    def body(c, carry):
        q = q_ref[0, pl.ds(c * 64, 64), :]
        kb = nk_s[pl.ds(c * 64, 128), :]
        d = lax.dot_general(q, kb, (((1,), (1,)), ((), ())),
                            preferred_element_type=jnp.float32)
        d = jnp.where(mb_s[pl.ds(c * 64, 64), :] != 0, NEG, d)
        m = jnp.max(d, axis=1, keepdims=True)
        e = jnp.exp(d - m)
        s = jnp.sum(e, axis=1, keepdims=True)
        vb = vx_s[pl.ds(c * 64, 128), :]
        o = lax.dot_general(e, vb, (((1,), (0,)), ((), ())),
                            preferred_element_type=jnp.float32) / s
        so_ref[0, pl.ds(c * 64, 64), :] = o
        sl_ref[0, pl.ds(c * 64, 64), :] = m + jnp.log(s)
        return carry

    lax.fori_loop(0, 32, body, 0, unroll=16)


def _attention(sqk, sv, trow, tcol):
    halo = lambda b, i: (b, (32 * i + 255) % 256, 0)
    cur = lambda b, i: (b, i, 0)
    halo4 = lambda b, i: (b, (32 * i + 255) % 256, 0, 0)
    cur4 = lambda b, i: (b, i, 0, 0)
    return pl.pallas_call(
        _att_body,
        grid=(B, TOTAL // 2048),
        in_specs=[
            pl.BlockSpec((1, 2048, D), cur),
            pl.BlockSpec((1, 64, D), halo),
            pl.BlockSpec((1, 2048, D), cur),
            pl.BlockSpec((1, 64, D), halo),
            pl.BlockSpec((1, 32, 1, 64), cur4),
            pl.BlockSpec((1, 1, 1, 64), halo4),
            pl.BlockSpec((1, 2048, 1), cur),
        ],
        out_specs=[pl.BlockSpec((1, 2048, D), cur),
                   pl.BlockSpec((1, 2048, 1), cur)],
        out_shape=[jax.ShapeDtypeStruct((B, TOTAL, D), jnp.float32),
                   jax.ShapeDtypeStruct((B, TOTAL, 1), jnp.float32)],
        scratch_shapes=[pltpu.VMEM((2112, D), jnp.float32),
                        pltpu.VMEM((2112, D), jnp.float32),
                        pltpu.VMEM((T, 128), jnp.float32)],
    )(sqk, sqk, sv, sv, trow, trow, tcol)


@functools.lru_cache(maxsize=None)
def _sc_unsort():
    return pl.kernel(
        _sc_unsort_body,
        compiler_params=pltpu.CompilerParams(needs_layout_passes=False, use_tc_tiling_on_sc=False),
        out_type=[jax.ShapeDtypeStruct((B * TOTAL, D), jnp.float32),
                  jax.ShapeDtypeStruct((B * TOTAL,), jnp.float32)],
        mesh=_mesh(),
        scratch_types=[pltpu.VMEM((SLOTS_PER_W // 128, 128), jnp.int32),
                       pltpu.VMEM((SLOTS_PER_W,), jnp.int32),
                       pltpu.VMEM((TOTAL,), jnp.float32),
                       pltpu.VMEM((1024, D), jnp.float32),
                       pltpu.VMEM((SLOTS_PER_W,), jnp.float32),
                       pltpu.SemaphoreType.DMA],
    )


def _sc_unsort_body(so_hbm, slog_hbm, pg_hbm, p_hbm, of_hbm, lf_hbm,
                    idx2, pv, slogv, rows, lout, sem):
    wid = lax.axis_index("s") * 2 + lax.axis_index("c")
    b = wid // 2
    base = wid * SLOTS_PER_W
    pltpu.sync_copy(pg_hbm.at[wid], idx2)
    pltpu.sync_copy(p_hbm.at[pl.ds(base, SLOTS_PER_W)], pv)
    pltpu.sync_copy(slog_hbm.at[b], slogv)

    def chunk(cc, carry):
        s0 = base + cc * 1024
        hs = [pltpu.async_copy(so_hbm.at[idx2.at[cc * 8 + kk]],
                               rows.at[pl.ds(kk * 128, 128)], sem)
              for kk in range(8)]
        for hh in hs:
            hh.wait()
        pltpu.sync_copy(rows, of_hbm.at[pl.ds(s0, 1024)])
        return carry

    lax.fori_loop(0, SLOTS_PER_W // 1024, chunk, 0)

    def lchunk(j, carry):
        idx16 = pv[pl.ds(j * 16, 16)]
        lout[pl.ds(j * 16, 16)] = plsc.load_gather(slogv, [idx16])
        return carry

    lax.fori_loop(0, SLOTS_PER_W // 16, lchunk, 0)
    pltpu.sync_copy(lout, lf_hbm.at[pl.ds(base, SLOTS_PER_W)])


def _combine_body(o_ref, l_ref, out_ref):
    ls = [l_ref[0, hh] for hh in range(NH)]
    m = ls[0]
    for hh in range(1, NH):
        m = jnp.maximum(m, ls[hh])
    es = [jnp.exp(lh - m) for lh in ls]
    s = es[0]
    for hh in range(1, NH):
        s = s + es[hh]
    acc = o_ref[0, 0] * es[0]
    for hh in range(1, NH):
        acc = acc + o_ref[0, hh] * es[hh]
    out_ref[0] = acc / s


def _combine(of, lf):
    return pl.pallas_call(
        _combine_body,
        grid=(B,),
        in_specs=[pl.BlockSpec((1, NH, T, D), lambda b: (b, 0, 0, 0)),
                  pl.BlockSpec((1, NH, T, 1), lambda b: (b, 0, 0, 0))],
        out_specs=pl.BlockSpec((1, T, D), lambda b: (b, 0, 0)),
        out_shape=jax.ShapeDtypeStruct((B, T, D), jnp.float32),
    )(of, lf)


def kernel(qk, k, v):
    del k
    qk2 = jnp.transpose(qk, (0, 2, 1, 3)).reshape(B, T, D)
    v2 = jnp.transpose(v, (0, 2, 1, 3)).reshape(B, T, D)
    rot = jax.random.normal(jax.random.key(42), (1, D, NH, NB // 2),
                            jnp.float32)[0].transpose(1, 0, 2)
    plocal3, p3, pg3 = _hash_pos(qk2, rot)
    st2, gst2 = _sc_scatter_st()(plocal3.reshape(B * NH, T))
    sqk, sv = _sc_gather_qkv()(qk2.reshape(B * T, D), v2.reshape(B * T, D),
                               gst2.reshape(NW, SLOTS_PER_W // 128, 128))
    stf = st2.reshape(B, TOTAL).astype(jnp.float32)
    so, slog = _attention(sqk.reshape(B, TOTAL, D), sv.reshape(B, TOTAL, D),
                          stf.reshape(B, TOTAL // 64, 1, 64),
                          stf.reshape(B, TOTAL, 1))
    of, lf = _sc_unsort()(so.reshape(B * TOTAL, D), slog.reshape(B, TOTAL),
                          pg3.reshape(NW, SLOTS_PER_W // 128, 128),
                          p3.reshape(B * TOTAL))
    return _combine(of.reshape(B, NH, T, D), lf.reshape(B, NH, T, 1))


# speedup vs baseline: 1.6536x; 1.4083x over previous
import functools

import jax
import jax.numpy as jnp
from jax import lax
from jax.experimental import pallas as pl
from jax.experimental.pallas import tpu as pltpu
from jax.experimental.pallas import tpu_sc as plsc

B = 16
T = 2048
D = 64
NH = 8
NB = 32
TOTAL = NH * T
NEG = -50000.0
NW = 32
ROWS_PER_W = (B * NH) // NW
SLOTS_PER_W = (B * TOTAL) // NW


def _hash_pos_body(qk_ref, rot_ref, plocal_ref, p_ref, pg_ref):
    b = pl.program_id(0)
    h = pl.program_id(1)
    qkb = qk_ref[0]
    r = rot_ref[0]
    rot = lax.dot_general(qkb, r, (((1,), (0,)), ((), ())),
                          preferred_element_type=jnp.float32)
    x = jnp.concatenate([rot, -rot], axis=1)
    m = jnp.max(x, axis=1, keepdims=True)
    iota = lax.broadcasted_iota(jnp.int32, (T, NB), 1)
    bi = jnp.min(jnp.where(x == m, iota, NB), axis=1, keepdims=True)
    oh = (iota == bi).astype(jnp.float32)
    tri = (lax.broadcasted_iota(jnp.int32, (128, 128), 0)
           >= lax.broadcasted_iota(jnp.int32, (128, 128), 1)).astype(jnp.float32)
    run = jnp.zeros((1, NB), jnp.float32)
    ranks = []
    for j in range(T // 128):
        blk = oh[j * 128:(j + 1) * 128, :]
        cg = lax.dot_general(tri, blk, (((1,), (0,)), ((), ())),
                             preferred_element_type=jnp.float32) + run
        ranks.append(jnp.sum(cg * blk, axis=1, keepdims=True) - 1.0)
        run = run + jnp.sum(blk, axis=0, keepdims=True)
    rank = jnp.concatenate(ranks, axis=0)
    startsel = jnp.sum(jnp.where(iota < bi, run, 0.0), axis=1, keepdims=True)
    plocal = (startsel + rank).astype(jnp.int32)
    plocal_ref[0] = plocal
    p_ref[0] = plocal + h * T
    pg_ref[0] = plocal + h * T + b * TOTAL


def _hash_pos(qk2, rot):
    return pl.pallas_call(
        _hash_pos_body,
        grid=(B, NH),
        in_specs=[
            pl.BlockSpec((1, T, D), lambda b, h: (b, 0, 0)),
            pl.BlockSpec((1, D, NB // 2), lambda b, h: (h, 0, 0)),
        ],
        out_specs=[pl.BlockSpec((1, T, 1), lambda b, h: (b * NH + h, 0, 0))] * 3,
        out_shape=[jax.ShapeDtypeStruct((B * NH, T, 1), jnp.int32)] * 3,
    )(qk2, rot)


@functools.lru_cache(maxsize=None)
def _mesh():
    return plsc.VectorSubcoreMesh(core_axis_name="c", subcore_axis_name="s")


@functools.lru_cache(maxsize=None)
def _sc_scatter_st():
    return pl.kernel(
        _sc_scatter_st_body,
        compiler_params=pltpu.CompilerParams(needs_layout_passes=False, use_tc_tiling_on_sc=False),
        out_type=[jax.ShapeDtypeStruct((B * NH, T), jnp.int32),
                  jax.ShapeDtypeStruct((B * NH, T), jnp.int32)],
        mesh=_mesh(),
        scratch_types=[pltpu.VMEM((T,), jnp.int32),
                       pltpu.VMEM((T,), jnp.int32),
                       pltpu.VMEM((T,), jnp.int32)],
    )


def _sc_scatter_st_body(pl_hbm, st_hbm, gst_hbm, pv, stv, gstv):
    wid = lax.axis_index("s") * 2 + lax.axis_index("c")

    def row_body(rr, carry):
        r = wid * ROWS_PER_W + rr
        b = r // NH
        pltpu.sync_copy(pl_hbm.at[r], pv)

        def inner(j, carry2):
            idx = pv[pl.ds(j * 16, 16)]
            t = j * 16 + lax.iota(jnp.int32, 16)
            plsc.store_scatter(stv, [idx], t)
            plsc.store_scatter(gstv, [idx], t + b * T)
            return carry2

        lax.fori_loop(0, T // 16, inner, 0)
        pltpu.sync_copy(stv, st_hbm.at[r])
        pltpu.sync_copy(gstv, gst_hbm.at[r])
        return carry

    lax.fori_loop(0, ROWS_PER_W, row_body, 0)


@functools.lru_cache(maxsize=None)
def _sc_gather_qkv():
    return pl.kernel(
        _sc_gather_qkv_body,
        compiler_params=pltpu.CompilerParams(needs_layout_passes=False, use_tc_tiling_on_sc=False),
        out_type=[jax.ShapeDtypeStruct((B * TOTAL, D), jnp.float32),
                  jax.ShapeDtypeStruct((B * TOTAL, D), jnp.float32)],
        mesh=_mesh(),
        scratch_types=[pltpu.VMEM((SLOTS_PER_W // 128, 128), jnp.int32),
                       pltpu.VMEM((1024, D), jnp.float32),
                       pltpu.SemaphoreType.DMA],
    )


def _sc_gather_qkv_body(qk_hbm, v_hbm, gst_hbm, sqk_hbm, sv_hbm, idx2, rows, sem):
    wid = lax.axis_index("s") * 2 + lax.axis_index("c")
    pltpu.sync_copy(gst_hbm.at[wid], idx2)
    base = wid * SLOTS_PER_W

    def chunk(cc, carry):
        s0 = base + cc * 1024
        hs = [pltpu.async_copy(qk_hbm.at[idx2.at[cc * 8 + kk]],
                               rows.at[pl.ds(kk * 128, 128)], sem)
              for kk in range(8)]
        for hh in hs:
            hh.wait()
        pltpu.sync_copy(rows, sqk_hbm.at[pl.ds(s0, 1024)])
        hs = [pltpu.async_copy(v_hbm.at[idx2.at[cc * 8 + kk]],
                               rows.at[pl.ds(kk * 128, 128)], sem)
              for kk in range(8)]
        for hh in hs:
            hh.wait()
        pltpu.sync_copy(rows, sv_hbm.at[pl.ds(s0, 1024)])
        return carry

    lax.fori_loop(0, SLOTS_PER_W // 1024, chunk, 0)


def _att_body(q_ref, kh_ref, v_ref, vh_ref, tr_ref, th_ref, tc_ref,
              so_ref, sl_ref, nk_s, vx_s, mb_s, d_s, rc_s):
    qall = q_ref[0]
    norm = jnp.sqrt(jnp.sum(qall * qall, axis=1, keepdims=True))
    nk_s[64:, :] = qall / jnp.maximum(norm, 1e-12)
    kh = kh_ref[0]
    nhh = jnp.sqrt(jnp.sum(kh * kh, axis=1, keepdims=True))
    nk_s[0:64, :] = kh / jnp.maximum(nhh, 1e-12)
    vx_s[64:, :] = v_ref[0]
    vx_s[0:64, :] = vh_ref[0]
    trow = tr_ref[...].reshape(32, 64)
    tprev = jnp.concatenate([th_ref[...].reshape(1, 64), trow[:31, :]], axis=0)
    tcol = tc_ref[0]
    tkb = jnp.concatenate([tprev, trow], axis=1)
    tkbig = jnp.broadcast_to(tkb[:, None, :], (32, 64, 128)).reshape(T, 128)
    mb_s[...] = (tcol == tkbig).astype(jnp.float32)

    for c in range(32):
        q = q_ref[0, c * 64:(c + 1) * 64, :]
        kb = nk_s[c * 64:c * 64 + 128, :]
        d_s[c * 64:(c + 1) * 64, :] = lax.dot_general(
            q, kb, (((1,), (1,)), ((), ())), preferred_element_type=jnp.float32)

    d = d_s[...]
    d = jnp.where(mb_s[...] != 0, NEG, d)
    m = jnp.max(d, axis=1, keepdims=True)
    e = jnp.exp(d - m)
    s = jnp.sum(e, axis=1, keepdims=True)
    d_s[...] = e
    rc_s[...] = 1.0 / s
    sl_ref[0] = m + jnp.log(s)

    for c in range(32):
        e = d_s[c * 64:(c + 1) * 64, :]
        vb = vx_s[c * 64:c * 64 + 128, :]
        o = lax.dot_general(e, vb, (((1,), (0,)), ((), ())),
                            preferred_element_type=jnp.float32)
        so_ref[0, c * 64:(c + 1) * 64, :] = o * rc_s[c * 64:(c + 1) * 64, :]


def _attention(sqk, sv, trow, tcol):
    halo = lambda b, i: (b, (32 * i + 255) % 256, 0)
    cur = lambda b, i: (b, i, 0)
    halo4 = lambda b, i: (b, (32 * i + 255) % 256, 0, 0)
    cur4 = lambda b, i: (b, i, 0, 0)
    return pl.pallas_call(
        _att_body,
        grid=(B, TOTAL // 2048),
        in_specs=[
            pl.BlockSpec((1, 2048, D), cur),
            pl.BlockSpec((1, 64, D), halo),
            pl.BlockSpec((1, 2048, D), cur),
            pl.BlockSpec((1, 64, D), halo),
            pl.BlockSpec((1, 32, 1, 64), cur4),
            pl.BlockSpec((1, 1, 1, 64), halo4),
            pl.BlockSpec((1, 2048, 1), cur),
        ],
        out_specs=[pl.BlockSpec((1, 2048, D), cur),
                   pl.BlockSpec((1, 2048, 1), cur)],
        out_shape=[jax.ShapeDtypeStruct((B, TOTAL, D), jnp.float32),
                   jax.ShapeDtypeStruct((B, TOTAL, 1), jnp.float32)],
        scratch_shapes=[pltpu.VMEM((2112, D), jnp.float32),
                        pltpu.VMEM((2112, D), jnp.float32),
                        pltpu.VMEM((T, 128), jnp.float32),
                        pltpu.VMEM((T, 128), jnp.float32),
                        pltpu.VMEM((T, 1), jnp.float32)],
    )(sqk, sqk, sv, sv, trow, trow, tcol)


@functools.lru_cache(maxsize=None)
def _sc_unsort():
    return pl.kernel(
        _sc_unsort_body,
        compiler_params=pltpu.CompilerParams(needs_layout_passes=False, use_tc_tiling_on_sc=False),
        out_type=[jax.ShapeDtypeStruct((B * TOTAL, D), jnp.float32),
                  jax.ShapeDtypeStruct((B * TOTAL,), jnp.float32)],
        mesh=_mesh(),
        scratch_types=[pltpu.VMEM((SLOTS_PER_W // 128, 128), jnp.int32),
                       pltpu.VMEM((SLOTS_PER_W,), jnp.int32),
                       pltpu.VMEM((TOTAL,), jnp.float32),
                       pltpu.VMEM((1024, D), jnp.float32),
                       pltpu.VMEM((SLOTS_PER_W,), jnp.float32),
                       pltpu.SemaphoreType.DMA],
    )


def _sc_unsort_body(so_hbm, slog_hbm, pg_hbm, p_hbm, of_hbm, lf_hbm,
                    idx2, pv, slogv, rows, lout, sem):
    wid = lax.axis_index("s") * 2 + lax.axis_index("c")
    b = wid // 2
    base = wid * SLOTS_PER_W
    pltpu.sync_copy(pg_hbm.at[wid], idx2)
    pltpu.sync_copy(p_hbm.at[pl.ds(base, SLOTS_PER_W)], pv)
    pltpu.sync_copy(slog_hbm.at[b], slogv)

    def chunk(cc, carry):
        s0 = base + cc * 1024
        hs = [pltpu.async_copy(so_hbm.at[idx2.at[cc * 8 + kk]],
                               rows.at[pl.ds(kk * 128, 128)], sem)
              for kk in range(8)]
        for hh in hs:
            hh.wait()
        pltpu.sync_copy(rows, of_hbm.at[pl.ds(s0, 1024)])
        return carry

    lax.fori_loop(0, SLOTS_PER_W // 1024, chunk, 0)

    def lchunk(j, carry):
        idx16 = pv[pl.ds(j * 16, 16)]
        lout[pl.ds(j * 16, 16)] = plsc.load_gather(slogv, [idx16])
        return carry

    lax.fori_loop(0, SLOTS_PER_W // 16, lchunk, 0)
    pltpu.sync_copy(lout, lf_hbm.at[pl.ds(base, SLOTS_PER_W)])


def _combine_body(o_ref, l_ref, out_ref):
    ls = [l_ref[0, hh] for hh in range(NH)]
    m = ls[0]
    for hh in range(1, NH):
        m = jnp.maximum(m, ls[hh])
    es = [jnp.exp(lh - m) for lh in ls]
    s = es[0]
    for hh in range(1, NH):
        s = s + es[hh]
    acc = o_ref[0, 0] * es[0]
    for hh in range(1, NH):
        acc = acc + o_ref[0, hh] * es[hh]
    out_ref[0] = acc / s


def _combine(of, lf):
    return pl.pallas_call(
        _combine_body,
        grid=(B,),
        in_specs=[pl.BlockSpec((1, NH, T, D), lambda b: (b, 0, 0, 0)),
                  pl.BlockSpec((1, NH, T, 1), lambda b: (b, 0, 0, 0))],
        out_specs=pl.BlockSpec((1, T, D), lambda b: (b, 0, 0)),
        out_shape=jax.ShapeDtypeStruct((B, T, D), jnp.float32),
    )(of, lf)


def kernel(qk, k, v):
    del k
    qk2 = jnp.transpose(qk, (0, 2, 1, 3)).reshape(B, T, D)
    v2 = jnp.transpose(v, (0, 2, 1, 3)).reshape(B, T, D)
    rot = jax.random.normal(jax.random.key(42), (1, D, NH, NB // 2),
                            jnp.float32)[0].transpose(1, 0, 2)
    plocal3, p3, pg3 = _hash_pos(qk2, rot)
    st2, gst2 = _sc_scatter_st()(plocal3.reshape(B * NH, T))
    sqk, sv = _sc_gather_qkv()(qk2.reshape(B * T, D), v2.reshape(B * T, D),
                               gst2.reshape(NW, SLOTS_PER_W // 128, 128))
    stf = st2.reshape(B, TOTAL).astype(jnp.float32)
    so, slog = _attention(sqk.reshape(B, TOTAL, D), sv.reshape(B, TOTAL, D),
                          stf.reshape(B, TOTAL // 64, 1, 64),
                          stf.reshape(B, TOTAL, 1))
    of, lf = _sc_unsort()(so.reshape(B * TOTAL, D), slog.reshape(B, TOTAL),
                          pg3.reshape(NW, SLOTS_PER_W // 128, 128),
                          p3.reshape(B * TOTAL))
    return _combine(of.reshape(B, NH, T, D), lf.reshape(B, NH, T, 1))


# speedup vs baseline: 1.9970x; 1.2077x over previous
import functools

import jax
import jax.numpy as jnp
from jax import lax
from jax.experimental import pallas as pl
from jax.experimental.pallas import tpu as pltpu
from jax.experimental.pallas import tpu_sc as plsc

B = 16
T = 2048
D = 64
NH = 8
NB = 32
TOTAL = NH * T
NEG = -50000.0
NW = 32
ROWS_PER_W = (B * NH) // NW
SLOTS_PER_W = (B * TOTAL) // NW


def _hash_pos_body(qk_ref, rot_ref, plocal_ref, p_ref, pg_ref):
    b = pl.program_id(0)
    qkb = qk_ref[0]
    rall = rot_ref[...]
    rotall = lax.dot_general(qkb, rall, (((1,), (0,)), ((), ())),
                             preferred_element_type=jnp.float32)
    tri = (lax.broadcasted_iota(jnp.int32, (128, 128), 0)
           >= lax.broadcasted_iota(jnp.int32, (128, 128), 1)).astype(jnp.float32)
    grp = (lax.broadcasted_iota(jnp.int32, (128, 4), 0) // NB
           == lax.broadcasted_iota(jnp.int32, (128, 4), 1)).astype(jnp.float32)
    lanemod = lax.broadcasted_iota(jnp.int32, (T, 128), 1) % NB
    iota32 = lax.broadcasted_iota(jnp.int32, (T, NB), 1)

    for g in range(2):
        bis = []
        for a in range(4):
            rh = rotall[:, (g * 4 + a) * 16:(g * 4 + a + 1) * 16]
            x = jnp.concatenate([rh, -rh], axis=1)
            m = jnp.max(x, axis=1, keepdims=True)
            bi = jnp.min(jnp.where(x == m, iota32, NB), axis=1, keepdims=True)
            bis.append(bi)
        bia = jnp.concatenate(
            [jnp.broadcast_to(bi, (T, NB)) for bi in bis], axis=1)
        ohall = (lanemod == bia).astype(jnp.float32)
        run = jnp.zeros((1, 128), jnp.float32)
        ranks = []
        for j in range(T // 128):
            blk = ohall[j * 128:(j + 1) * 128, :]
            cg = lax.dot_general(tri, blk, (((1,), (0,)), ((), ())),
                                 preferred_element_type=jnp.float32) + run
            ranks.append(lax.dot_general(cg * blk, grp, (((1,), (0,)), ((), ())),
                                         preferred_element_type=jnp.float32) - 1.0)
            run = run + jnp.sum(blk, axis=0, keepdims=True)
        rank4 = jnp.concatenate(ranks, axis=0)
        w = jnp.where(lanemod < bia, run, 0.0)
        start4 = lax.dot_general(w, grp, (((1,), (0,)), ((), ())),
                                 preferred_element_type=jnp.float32)
        plocal4 = (start4 + rank4).astype(jnp.int32)
        hoff = (lax.broadcasted_iota(jnp.int32, (T, 4), 1) + g * 4) * T
        plocal_ref[0, :, g * 4:(g + 1) * 4] = plocal4
        p_ref[0, :, g * 4:(g + 1) * 4] = plocal4 + hoff
        pg_ref[0, :, g * 4:(g + 1) * 4] = plocal4 + hoff + b * TOTAL


def _hash_pos(qk2, rotall):
    return pl.pallas_call(
        _hash_pos_body,
        grid=(B,),
        in_specs=[
            pl.BlockSpec((1, T, D), lambda b: (b, 0, 0)),
            pl.BlockSpec((D, 128), lambda b: (0, 0)),
        ],
        out_specs=[pl.BlockSpec((1, T, NH), lambda b: (b, 0, 0))] * 3,
        out_shape=[jax.ShapeDtypeStruct((B, T, NH), jnp.int32)] * 3,
    )(qk2, rotall)


@functools.lru_cache(maxsize=None)
def _mesh():
    return plsc.VectorSubcoreMesh(core_axis_name="c", subcore_axis_name="s")


@functools.lru_cache(maxsize=None)
def _sc_scatter_st():
    return pl.kernel(
        _sc_scatter_st_body,
        compiler_params=pltpu.CompilerParams(needs_layout_passes=False, use_tc_tiling_on_sc=False),
        out_type=[jax.ShapeDtypeStruct((B * NH, T), jnp.int32),
                  jax.ShapeDtypeStruct((B * NH, T), jnp.int32)],
        mesh=_mesh(),
        scratch_types=[pltpu.VMEM((T,), jnp.int32),
                       pltpu.VMEM((T,), jnp.int32),
                       pltpu.VMEM((T,), jnp.int32)],
    )


def _sc_scatter_st_body(pl_hbm, st_hbm, gst_hbm, pv, stv, gstv):
    wid = lax.axis_index("s") * 2 + lax.axis_index("c")

    def row_body(rr, carry):
        r = wid * ROWS_PER_W + rr
        b = r // NH
        pltpu.sync_copy(pl_hbm.at[r], pv)

        def inner(j, carry2):
            idx = pv[pl.ds(j * 16, 16)]
            t = j * 16 + lax.iota(jnp.int32, 16)
            plsc.store_scatter(stv, [idx], t)
            plsc.store_scatter(gstv, [idx], t + b * T)
            return carry2

        lax.fori_loop(0, T // 16, inner, 0)
        pltpu.sync_copy(stv, st_hbm.at[r])
        pltpu.sync_copy(gstv, gst_hbm.at[r])
        return carry

    lax.fori_loop(0, ROWS_PER_W, row_body, 0)


@functools.lru_cache(maxsize=None)
def _sc_gather_qkv():
    return pl.kernel(
        _sc_gather_qkv_body,
        compiler_params=pltpu.CompilerParams(needs_layout_passes=False, use_tc_tiling_on_sc=False),
        out_type=[jax.ShapeDtypeStruct((B * TOTAL, D), jnp.float32),
                  jax.ShapeDtypeStruct((B * TOTAL, D), jnp.float32)],
        mesh=_mesh(),
        scratch_types=[pltpu.VMEM((SLOTS_PER_W // 128, 128), jnp.int32),
                       pltpu.VMEM((1024, D), jnp.float32),
                       pltpu.SemaphoreType.DMA],
    )


def _sc_gather_qkv_body(qk_hbm, v_hbm, gst_hbm, sqk_hbm, sv_hbm, idx2, rows, sem):
    wid = lax.axis_index("s") * 2 + lax.axis_index("c")
    pltpu.sync_copy(gst_hbm.at[wid], idx2)
    base = wid * SLOTS_PER_W

    def chunk(cc, carry):
        s0 = base + cc * 1024
        hs = [pltpu.async_copy(qk_hbm.at[idx2.at[cc * 8 + kk]],
                               rows.at[pl.ds(kk * 128, 128)], sem)
              for kk in range(8)]
        for hh in hs:
            hh.wait()
        pltpu.sync_copy(rows, sqk_hbm.at[pl.ds(s0, 1024)])
        hs = [pltpu.async_copy(v_hbm.at[idx2.at[cc * 8 + kk]],
                               rows.at[pl.ds(kk * 128, 128)], sem)
              for kk in range(8)]
        for hh in hs:
            hh.wait()
        pltpu.sync_copy(rows, sv_hbm.at[pl.ds(s0, 1024)])
        return carry

    lax.fori_loop(0, SLOTS_PER_W // 1024, chunk, 0)


def _att_body(q_ref, kh_ref, v_ref, vh_ref, tr_ref, th_ref, tc_ref,
              so_ref, sl_ref, nk_s, vx_s, mb_s, d_s, rc_s):
    qall = q_ref[0]
    norm = jnp.sqrt(jnp.sum(qall * qall, axis=1, keepdims=True))
    nk_s[64:, :] = qall / jnp.maximum(norm, 1e-12)
    kh = kh_ref[0]
    nhh = jnp.sqrt(jnp.sum(kh * kh, axis=1, keepdims=True))
    nk_s[0:64, :] = kh / jnp.maximum(nhh, 1e-12)
    vx_s[64:, :] = v_ref[0]
    vx_s[0:64, :] = vh_ref[0]
    trow = tr_ref[...].reshape(32, 64)
    tprev = jnp.concatenate([th_ref[...].reshape(1, 64), trow[:31, :]], axis=0)
    tcol = tc_ref[0]
    tkb = jnp.concatenate([tprev, trow], axis=1)
    tkbig = jnp.broadcast_to(tkb[:, None, :], (32, 64, 128)).reshape(T, 128)
    mb_s[...] = (tcol == tkbig).astype(jnp.float32)

    for c in range(32):
        q = q_ref[0, c * 64:(c + 1) * 64, :]
        kb = nk_s[c * 64:c * 64 + 128, :]
        d_s[c * 64:(c + 1) * 64, :] = lax.dot_general(
            q, kb, (((1,), (1,)), ((), ())), preferred_element_type=jnp.float32)

    d = d_s[...]
    d = jnp.where(mb_s[...] != 0, NEG, d)
    m = jnp.max(d, axis=1, keepdims=True)
    e = jnp.exp(d - m)
    s = jnp.sum(e, axis=1, keepdims=True)
    d_s[...] = e
    rc_s[...] = 1.0 / s
    sl_ref[0] = m + jnp.log(s)

    for c in range(32):
        e = d_s[c * 64:(c + 1) * 64, :]
        vb = vx_s[c * 64:c * 64 + 128, :]
        o = lax.dot_general(e, vb, (((1,), (0,)), ((), ())),
                            preferred_element_type=jnp.float32)
        so_ref[0, c * 64:(c + 1) * 64, :] = o * rc_s[c * 64:(c + 1) * 64, :]


def _attention(sqk, sv, trow, tcol):
    halo = lambda b, i: (b, (32 * i + 255) % 256, 0)
    cur = lambda b, i: (b, i, 0)
    halo4 = lambda b, i: (b, (32 * i + 255) % 256, 0, 0)
    cur4 = lambda b, i: (b, i, 0, 0)
    return pl.pallas_call(
        _att_body,
        grid=(B, TOTAL // 2048),
        in_specs=[
            pl.BlockSpec((1, 2048, D), cur),
            pl.BlockSpec((1, 64, D), halo),
            pl.BlockSpec((1, 2048, D), cur),
            pl.BlockSpec((1, 64, D), halo),
            pl.BlockSpec((1, 32, 1, 64), cur4),
            pl.BlockSpec((1, 1, 1, 64), halo4),
            pl.BlockSpec((1, 2048, 1), cur),
        ],
        out_specs=[pl.BlockSpec((1, 2048, D), cur),
                   pl.BlockSpec((1, 2048, 1), cur)],
        out_shape=[jax.ShapeDtypeStruct((B, TOTAL, D), jnp.float32),
                   jax.ShapeDtypeStruct((B, TOTAL, 1), jnp.float32)],
        scratch_shapes=[pltpu.VMEM((2112, D), jnp.float32),
                        pltpu.VMEM((2112, D), jnp.float32),
                        pltpu.VMEM((T, 128), jnp.float32),
                        pltpu.VMEM((T, 128), jnp.float32),
                        pltpu.VMEM((T, 1), jnp.float32)],
    )(sqk, sqk, sv, sv, trow, trow, tcol)


@functools.lru_cache(maxsize=None)
def _sc_unsort():
    return pl.kernel(
        _sc_unsort_body,
        compiler_params=pltpu.CompilerParams(needs_layout_passes=False, use_tc_tiling_on_sc=False),
        out_type=[jax.ShapeDtypeStruct((B * TOTAL, D), jnp.float32),
                  jax.ShapeDtypeStruct((B * TOTAL,), jnp.float32)],
        mesh=_mesh(),
        scratch_types=[pltpu.VMEM((SLOTS_PER_W // 128, 128), jnp.int32),
                       pltpu.VMEM((SLOTS_PER_W,), jnp.int32),
                       pltpu.VMEM((TOTAL,), jnp.float32),
                       pltpu.VMEM((1024, D), jnp.float32),
                       pltpu.VMEM((SLOTS_PER_W,), jnp.float32),
                       pltpu.SemaphoreType.DMA],
    )


def _sc_unsort_body(so_hbm, slog_hbm, pg_hbm, p_hbm, of_hbm, lf_hbm,
                    idx2, pv, slogv, rows, lout, sem):
    wid = lax.axis_index("s") * 2 + lax.axis_index("c")
    b = wid // 2
    base = wid * SLOTS_PER_W
    pltpu.sync_copy(pg_hbm.at[wid], idx2)
    pltpu.sync_copy(p_hbm.at[pl.ds(base, SLOTS_PER_W)], pv)
    pltpu.sync_copy(slog_hbm.at[b], slogv)

    def chunk(cc, carry):
        s0 = base + cc * 1024
        hs = [pltpu.async_copy(so_hbm.at[idx2.at[cc * 8 + kk]],
                               rows.at[pl.ds(kk * 128, 128)], sem)
              for kk in range(8)]
        for hh in hs:
            hh.wait()
        pltpu.sync_copy(rows, of_hbm.at[pl.ds(s0, 1024)])
        return carry

    lax.fori_loop(0, SLOTS_PER_W // 1024, chunk, 0)

    def lchunk(j, carry):
        idx16 = pv[pl.ds(j * 16, 16)]
        lout[pl.ds(j * 16, 16)] = plsc.load_gather(slogv, [idx16])
        return carry

    lax.fori_loop(0, SLOTS_PER_W // 16, lchunk, 0)
    pltpu.sync_copy(lout, lf_hbm.at[pl.ds(base, SLOTS_PER_W)])


def _combine_body(o_ref, l_ref, out_ref):
    ls = [l_ref[0, hh] for hh in range(NH)]
    m = ls[0]
    for hh in range(1, NH):
        m = jnp.maximum(m, ls[hh])
    es = [jnp.exp(lh - m) for lh in ls]
    s = es[0]
    for hh in range(1, NH):
        s = s + es[hh]
    acc = o_ref[0, 0] * es[0]
    for hh in range(1, NH):
        acc = acc + o_ref[0, hh] * es[hh]
    out_ref[0] = acc / s


def _combine(of, lf):
    return pl.pallas_call(
        _combine_body,
        grid=(B,),
        in_specs=[pl.BlockSpec((1, NH, T, D), lambda b: (b, 0, 0, 0)),
                  pl.BlockSpec((1, NH, T, 1), lambda b: (b, 0, 0, 0))],
        out_specs=pl.BlockSpec((1, T, D), lambda b: (b, 0, 0)),
        out_shape=jax.ShapeDtypeStruct((B, T, D), jnp.float32),
    )(of, lf)


def kernel(qk, k, v):
    del k
    qk2 = jnp.transpose(qk, (0, 2, 1, 3)).reshape(B, T, D)
    v2 = jnp.transpose(v, (0, 2, 1, 3)).reshape(B, T, D)
    rotall = jax.random.normal(jax.random.key(42), (1, D, NH, NB // 2),
                               jnp.float32)[0].reshape(D, NH * (NB // 2))
    plocal3, p3, pg3 = _hash_pos(qk2, rotall)
    plocal3 = jnp.transpose(plocal3, (0, 2, 1))
    p3 = jnp.transpose(p3, (0, 2, 1))
    pg3 = jnp.transpose(pg3, (0, 2, 1))
    st2, gst2 = _sc_scatter_st()(plocal3.reshape(B * NH, T))
    sqk, sv = _sc_gather_qkv()(qk2.reshape(B * T, D), v2.reshape(B * T, D),
                               gst2.reshape(NW, SLOTS_PER_W // 128, 128))
    stf = st2.reshape(B, TOTAL).astype(jnp.float32)
    so, slog = _attention(sqk.reshape(B, TOTAL, D), sv.reshape(B, TOTAL, D),
                          stf.reshape(B, TOTAL // 64, 1, 64),
                          stf.reshape(B, TOTAL, 1))
    of, lf = _sc_unsort()(so.reshape(B * TOTAL, D), slog.reshape(B, TOTAL),
                          pg3.reshape(NW, SLOTS_PER_W // 128, 128),
                          p3.reshape(B * TOTAL))
    return _combine(of.reshape(B, NH, T, D), lf.reshape(B, NH, T, 1))


# speedup vs baseline: 2.0030x; 1.0030x over previous
import functools

import jax
import jax.numpy as jnp
from jax import lax
from jax.experimental import pallas as pl
from jax.experimental.pallas import tpu as pltpu
from jax.experimental.pallas import tpu_sc as plsc

B = 16
T = 2048
D = 64
NH = 8
NB = 32
TOTAL = NH * T
NEG = -50000.0
NW = 32
ROWS_PER_W = (B * NH) // NW
SLOTS_PER_W = (B * TOTAL) // NW


def _hash_pos_body(qk_ref, rot_ref, plocal_ref, p_ref, pg_ref):
    b = pl.program_id(0)
    qkb = qk_ref[0]
    rall = rot_ref[...]
    rotall = lax.dot_general(qkb, rall, (((1,), (0,)), ((), ())),
                             preferred_element_type=jnp.float32)
    tri = (lax.broadcasted_iota(jnp.int32, (128, 128), 0)
           >= lax.broadcasted_iota(jnp.int32, (128, 128), 1)).astype(jnp.float32)
    grp = (lax.broadcasted_iota(jnp.int32, (128, 4), 0) // NB
           == lax.broadcasted_iota(jnp.int32, (128, 4), 1)).astype(jnp.float32)
    lanemod = lax.broadcasted_iota(jnp.int32, (T, 128), 1) % NB
    iota32 = lax.broadcasted_iota(jnp.int32, (T, NB), 1)

    for g in range(2):
        bis = []
        for a in range(4):
            rh = rotall[:, (g * 4 + a) * 16:(g * 4 + a + 1) * 16]
            x = jnp.concatenate([rh, -rh], axis=1)
            m = jnp.max(x, axis=1, keepdims=True)
            bi = jnp.min(jnp.where(x == m, iota32, NB), axis=1, keepdims=True)
            bis.append(bi)
        bia = jnp.concatenate(
            [jnp.broadcast_to(bi, (T, NB)) for bi in bis], axis=1)
        ohall = (lanemod == bia).astype(jnp.float32)
        run = jnp.zeros((1, 128), jnp.float32)
        ranks = []
        for j in range(T // 128):
            blk = ohall[j * 128:(j + 1) * 128, :]
            cg = lax.dot_general(tri, blk, (((1,), (0,)), ((), ())),
                                 preferred_element_type=jnp.float32) + run
            ranks.append(lax.dot_general(cg * blk, grp, (((1,), (0,)), ((), ())),
                                         preferred_element_type=jnp.float32) - 1.0)
            run = run + jnp.sum(blk, axis=0, keepdims=True)
        rank4 = jnp.concatenate(ranks, axis=0)
        w = jnp.where(lanemod < bia, run, 0.0)
        start4 = lax.dot_general(w, grp, (((1,), (0,)), ((), ())),
                                 preferred_element_type=jnp.float32)
        plocal4 = (start4 + rank4).astype(jnp.int32)
        hoff = (lax.broadcasted_iota(jnp.int32, (T, 4), 1) + g * 4) * T
        plocal_ref[0, :, g * 4:(g + 1) * 4] = plocal4
        p_ref[0, :, g * 4:(g + 1) * 4] = plocal4 + hoff
        pg_ref[0, :, g * 4:(g + 1) * 4] = plocal4 + hoff + b * TOTAL


def _hash_pos(qk2, rotall):
    return pl.pallas_call(
        _hash_pos_body,
        grid=(B,),
        in_specs=[
            pl.BlockSpec((1, T, D), lambda b: (b, 0, 0)),
            pl.BlockSpec((D, 128), lambda b: (0, 0)),
        ],
        out_specs=[pl.BlockSpec((1, T, NH), lambda b: (b, 0, 0))] * 3,
        out_shape=[jax.ShapeDtypeStruct((B, T, NH), jnp.int32)] * 3,
    )(qk2, rotall)


@functools.lru_cache(maxsize=None)
def _mesh():
    return plsc.VectorSubcoreMesh(core_axis_name="c", subcore_axis_name="s")


@functools.lru_cache(maxsize=None)
def _sc_scatter_st():
    return pl.kernel(
        _sc_scatter_st_body,
        compiler_params=pltpu.CompilerParams(needs_layout_passes=False, use_tc_tiling_on_sc=False),
        out_type=[jax.ShapeDtypeStruct((B * NH, T), jnp.int32),
                  jax.ShapeDtypeStruct((B * NH, T), jnp.int32)],
        mesh=_mesh(),
        scratch_types=[pltpu.VMEM((T,), jnp.int32),
                       pltpu.VMEM((T,), jnp.int32),
                       pltpu.VMEM((T,), jnp.int32)],
    )


def _sc_scatter_st_body(pl_hbm, st_hbm, gst_hbm, pv, stv, gstv):
    wid = lax.axis_index("s") * 2 + lax.axis_index("c")

    def row_body(rr, carry):
        r = wid * ROWS_PER_W + rr
        b = r // NH
        pltpu.sync_copy(pl_hbm.at[r], pv)

        def inner(j, carry2):
            idx = pv[pl.ds(j * 16, 16)]
            t = j * 16 + lax.iota(jnp.int32, 16)
            plsc.store_scatter(stv, [idx], t)
            plsc.store_scatter(gstv, [idx], t + b * T)
            return carry2

        lax.fori_loop(0, T // 16, inner, 0)
        pltpu.sync_copy(stv, st_hbm.at[r])
        pltpu.sync_copy(gstv, gst_hbm.at[r])
        return carry

    lax.fori_loop(0, ROWS_PER_W, row_body, 0)


@functools.lru_cache(maxsize=None)
def _sc_gather_qkv():
    return pl.kernel(
        _sc_gather_qkv_body,
        compiler_params=pltpu.CompilerParams(needs_layout_passes=False, use_tc_tiling_on_sc=False),
        out_type=[jax.ShapeDtypeStruct((B * TOTAL, D), jnp.float32),
                  jax.ShapeDtypeStruct((B * TOTAL, D), jnp.float32)],
        mesh=_mesh(),
        scratch_types=[pltpu.VMEM((SLOTS_PER_W // 128, 128), jnp.int32),
                       pltpu.VMEM((1024, D), jnp.float32),
                       pltpu.SemaphoreType.DMA],
    )


def _sc_gather_qkv_body(qk_hbm, v_hbm, gst_hbm, sqk_hbm, sv_hbm, idx2, rows, sem):
    wid = lax.axis_index("s") * 2 + lax.axis_index("c")
    pltpu.sync_copy(gst_hbm.at[wid], idx2)
    base = wid * SLOTS_PER_W

    def chunk(cc, carry):
        s0 = base + cc * 1024
        hs = [pltpu.async_copy(qk_hbm.at[idx2.at[cc * 8 + kk]],
                               rows.at[pl.ds(kk * 128, 128)], sem)
              for kk in range(8)]
        for hh in hs:
            hh.wait()
        pltpu.sync_copy(rows, sqk_hbm.at[pl.ds(s0, 1024)])
        hs = [pltpu.async_copy(v_hbm.at[idx2.at[cc * 8 + kk]],
                               rows.at[pl.ds(kk * 128, 128)], sem)
              for kk in range(8)]
        for hh in hs:
            hh.wait()
        pltpu.sync_copy(rows, sv_hbm.at[pl.ds(s0, 1024)])
        return carry

    lax.fori_loop(0, SLOTS_PER_W // 1024, chunk, 0)


def _att_body(q_ref, kh_ref, v_ref, vh_ref, tr_ref, th_ref, tc_ref,
              so_ref, sl_ref, nk_s, vx_s, mb_s, d_s, rc_s):
    qall = q_ref[0]
    nk_s[64:, :] = qall * lax.rsqrt(
        jnp.maximum(jnp.sum(qall * qall, axis=1, keepdims=True), 1e-24))
    kh = kh_ref[0]
    nk_s[0:64, :] = kh * lax.rsqrt(
        jnp.maximum(jnp.sum(kh * kh, axis=1, keepdims=True), 1e-24))
    vx_s[64:, :] = v_ref[0]
    vx_s[0:64, :] = vh_ref[0]
    trow = tr_ref[...].reshape(32, 64)
    tprev = jnp.concatenate([th_ref[...].reshape(1, 64), trow[:31, :]], axis=0)
    tcol = tc_ref[0]
    tkb = jnp.concatenate([tprev, trow], axis=1)
    tkbig = jnp.broadcast_to(tkb[:, None, :], (32, 64, 128)).reshape(T, 128)
    mb_s[...] = (tcol == tkbig).astype(jnp.float32)

    for c in range(32):
        q = q_ref[0, c * 64:(c + 1) * 64, :].astype(jnp.bfloat16)
        kb = nk_s[c * 64:c * 64 + 128, :].astype(jnp.bfloat16)
        d_s[c * 64:(c + 1) * 64, :] = lax.dot_general(
            q, kb, (((1,), (1,)), ((), ())), preferred_element_type=jnp.float32)

    d = d_s[...]
    d = jnp.where(mb_s[...] != 0, NEG, d)
    m = jnp.max(d, axis=1, keepdims=True)
    e = jnp.exp(d - m)
    s = jnp.sum(e, axis=1, keepdims=True)
    d_s[...] = e
    rc_s[...] = 1.0 / s
    sl_ref[0] = m + jnp.log(s)

    for c in range(32):
        e = d_s[c * 64:(c + 1) * 64, :].astype(jnp.bfloat16)
        vb = vx_s[c * 64:c * 64 + 128, :].astype(jnp.bfloat16)
        o = lax.dot_general(e, vb, (((1,), (0,)), ((), ())),
                            preferred_element_type=jnp.float32)
        so_ref[0, c * 64:(c + 1) * 64, :] = o * rc_s[c * 64:(c + 1) * 64, :]


def _attention(sqk, sv, trow, tcol):
    halo = lambda b, i: (b, (32 * i + 255) % 256, 0)
    cur = lambda b, i: (b, i, 0)
    halo4 = lambda b, i: (b, (32 * i + 255) % 256, 0, 0)
    cur4 = lambda b, i: (b, i, 0, 0)
    return pl.pallas_call(
        _att_body,
        grid=(B, TOTAL // 2048),
        in_specs=[
            pl.BlockSpec((1, 2048, D), cur),
            pl.BlockSpec((1, 64, D), halo),
            pl.BlockSpec((1, 2048, D), cur),
            pl.BlockSpec((1, 64, D), halo),
            pl.BlockSpec((1, 32, 1, 64), cur4),
            pl.BlockSpec((1, 1, 1, 64), halo4),
            pl.BlockSpec((1, 2048, 1), cur),
        ],
        out_specs=[pl.BlockSpec((1, 2048, D), cur),
                   pl.BlockSpec((1, 2048, 1), cur)],
        out_shape=[jax.ShapeDtypeStruct((B, TOTAL, D), jnp.float32),
                   jax.ShapeDtypeStruct((B, TOTAL, 1), jnp.float32)],
        scratch_shapes=[pltpu.VMEM((2112, D), jnp.float32),
                        pltpu.VMEM((2112, D), jnp.float32),
                        pltpu.VMEM((T, 128), jnp.float32),
                        pltpu.VMEM((T, 128), jnp.float32),
                        pltpu.VMEM((T, 1), jnp.float32)],
    )(sqk, sqk, sv, sv, trow, trow, tcol)


@functools.lru_cache(maxsize=None)
def _sc_unsort():
    return pl.kernel(
        _sc_unsort_body,
        compiler_params=pltpu.CompilerParams(needs_layout_passes=False, use_tc_tiling_on_sc=False),
        out_type=[jax.ShapeDtypeStruct((B * TOTAL, D), jnp.float32),
                  jax.ShapeDtypeStruct((B * TOTAL,), jnp.float32)],
        mesh=_mesh(),
        scratch_types=[pltpu.VMEM((SLOTS_PER_W // 128, 128), jnp.int32),
                       pltpu.VMEM((SLOTS_PER_W,), jnp.int32),
                       pltpu.VMEM((TOTAL,), jnp.float32),
                       pltpu.VMEM((1024, D), jnp.float32),
                       pltpu.VMEM((SLOTS_PER_W,), jnp.float32),
                       pltpu.SemaphoreType.DMA],
    )


def _sc_unsort_body(so_hbm, slog_hbm, pg_hbm, p_hbm, of_hbm, lf_hbm,
                    idx2, pv, slogv, rows, lout, sem):
    wid = lax.axis_index("s") * 2 + lax.axis_index("c")
    b = wid // 2
    base = wid * SLOTS_PER_W
    pltpu.sync_copy(pg_hbm.at[wid], idx2)
    pltpu.sync_copy(p_hbm.at[pl.ds(base, SLOTS_PER_W)], pv)
    pltpu.sync_copy(slog_hbm.at[b], slogv)

    def chunk(cc, carry):
        s0 = base + cc * 1024
        hs = [pltpu.async_copy(so_hbm.at[idx2.at[cc * 8 + kk]],
                               rows.at[pl.ds(kk * 128, 128)], sem)
              for kk in range(8)]
        for hh in hs:
            hh.wait()
        pltpu.sync_copy(rows, of_hbm.at[pl.ds(s0, 1024)])
        return carry

    lax.fori_loop(0, SLOTS_PER_W // 1024, chunk, 0)

    def lchunk(j, carry):
        idx16 = pv[pl.ds(j * 16, 16)]
        lout[pl.ds(j * 16, 16)] = plsc.load_gather(slogv, [idx16])
        return carry

    lax.fori_loop(0, SLOTS_PER_W // 16, lchunk, 0)
    pltpu.sync_copy(lout, lf_hbm.at[pl.ds(base, SLOTS_PER_W)])


def _combine_body(o_ref, l_ref, out_ref):
    ls = [l_ref[0, hh] for hh in range(NH)]
    m = ls[0]
    for hh in range(1, NH):
        m = jnp.maximum(m, ls[hh])
    es = [jnp.exp(lh - m) for lh in ls]
    s = es[0]
    for hh in range(1, NH):
        s = s + es[hh]
    acc = o_ref[0, 0] * es[0]
    for hh in range(1, NH):
        acc = acc + o_ref[0, hh] * es[hh]
    out_ref[0] = acc / s


def _combine(of, lf):
    return pl.pallas_call(
        _combine_body,
        grid=(B,),
        in_specs=[pl.BlockSpec((1, NH, T, D), lambda b: (b, 0, 0, 0)),
                  pl.BlockSpec((1, NH, T, 1), lambda b: (b, 0, 0, 0))],
        out_specs=pl.BlockSpec((1, T, D), lambda b: (b, 0, 0)),
        out_shape=jax.ShapeDtypeStruct((B, T, D), jnp.float32),
    )(of, lf)


def kernel(qk, k, v):
    del k
    qk2 = jnp.transpose(qk, (0, 2, 1, 3)).reshape(B, T, D)
    v2 = jnp.transpose(v, (0, 2, 1, 3)).reshape(B, T, D)
    rotall = jax.random.normal(jax.random.key(42), (1, D, NH, NB // 2),
                               jnp.float32)[0].reshape(D, NH * (NB // 2))
    plocal3, p3, pg3 = _hash_pos(qk2, rotall)
    plocal3 = jnp.transpose(plocal3, (0, 2, 1))
    p3 = jnp.transpose(p3, (0, 2, 1))
    pg3 = jnp.transpose(pg3, (0, 2, 1))
    st2, gst2 = _sc_scatter_st()(plocal3.reshape(B * NH, T))
    sqk, sv = _sc_gather_qkv()(qk2.reshape(B * T, D), v2.reshape(B * T, D),
                               gst2.reshape(NW, SLOTS_PER_W // 128, 128))
    stf = st2.reshape(B, TOTAL).astype(jnp.float32)
    so, slog = _attention(sqk.reshape(B, TOTAL, D), sv.reshape(B, TOTAL, D),
                          stf.reshape(B, TOTAL // 64, 1, 64),
                          stf.reshape(B, TOTAL, 1))
    of, lf = _sc_unsort()(so.reshape(B * TOTAL, D), slog.reshape(B, TOTAL),
                          pg3.reshape(NW, SLOTS_PER_W // 128, 128),
                          p3.reshape(B * TOTAL))
    return _combine(of.reshape(B, NH, T, D), lf.reshape(B, NH, T, 1))
